# baseline (speedup 1.0000x reference)
import functools

import jax
import jax.numpy as jnp
import numpy as np
from jax import lax
from jax.experimental import pallas as pl
from jax.experimental.pallas import tpu as pltpu
from jax.experimental.pallas import tpu_sc as plsc

_K = 64
_B = 128
_N = 32768
_L = 16
_VPB = 4
_BLK = _L * _VPB
_NBLK = _N // _BLK
_NBINS = 4096
_SIGN = np.int32(-2147483648)
_NOSIGN = np.int32(0x7FFFFFFF)


def _build_sc_select():
    info = plsc.get_sparse_core_info()
    NC, NS = info.num_cores, info.num_subcores
    NW = NC * NS
    RPW = _B // NW

    mesh = plsc.VectorSubcoreMesh(core_axis_name="c", subcore_axis_name="s")

    @functools.partial(
        pl.kernel,
        out_type=[
            jax.ShapeDtypeStruct((NW, _L), jnp.float32),
            jax.ShapeDtypeStruct((NW, _L), jnp.int32),
        ],
        mesh=mesh,
        compiler_params=pltpu.CompilerParams(needs_layout_passes=False),
        scratch_types=[
            pltpu.VMEM((_N,), jnp.float32),
            pltpu.VMEM((_N,), jnp.float32),
            pltpu.VMEM((_NBINS,), jnp.int32),
            pltpu.VMEM((_L,), jnp.float32),
            pltpu.VMEM((_L,), jnp.int32),
            pltpu.SemaphoreType.DMA,
        ],
    )
    def sc_select(scores_hbm, thr_hbm, tie_hbm, keya_v, keyb_v, hist_v,
                  sthr_v, stie_v, dma_sem):
        wid = lax.axis_index("s") * NC + lax.axis_index("c")

        zi = jnp.zeros((_L,), jnp.int32)
        oi = jnp.ones((_L,), jnp.int32)
        lane = lax.broadcasted_iota(jnp.int32, (_L,), 0)
        stage_thr = jnp.zeros((_L,), jnp.float32)
        stage_tie = zi

        key_bufs = [keya_v, keyb_v]

        def hist_clear(nwords):
            @plsc.parallel_loop(0, nwords // _L, unroll=4)
            def _clr(i):
                hist_v[pl.ds(i * _L, _L)] = zi

        def hist_scan(nbins, need):
            def cond(c):
                return jnp.logical_not(c[2])

            def body(c):
                blk4, cum, found = c
                base = blk4 * 4 * _L
                s = hist_v[pl.ds(base, _L)]
                for j in range(1, 4):
                    s = s + hist_v[pl.ds(base + j * _L, _L)]
                bs = jnp.sum(s)
                hit = cum + bs >= need
                blk42 = jnp.where(hit, blk4, blk4 - 1)
                cum2 = jnp.where(hit, cum, cum + bs)
                return blk42, cum2, hit

            init = (jnp.int32(nbins // (4 * _L) - 1), jnp.int32(0),
                    jnp.bool_(False))
            blk4, cum, _ = lax.while_loop(cond, body, init)

            base = blk4 * 4 * _L
            run = cum
            taken = jnp.bool_(False)
            bsp = zi
            ncs = jnp.int32(0)
            bcnt = jnp.int32(0)
            for j in range(3, -1, -1):
                v = hist_v[pl.ds(base + j * _L, _L)]
                bs = jnp.sum(v)
                fw = plsc.cumsum(v)
                cnd = (run + bs - fw + v) >= need
                pc = plsc.all_reduce_population_count(cnd)
                bsp2 = base + j * _L + pc - 1
                oh = lane == (pc - 1)
                fw_b = jnp.sum(jnp.where(oh, fw, zi))
                v_b = jnp.sum(jnp.where(oh, v, zi))
                ncs2 = need - (run + bs - fw_b)
                hit_j = run + bs >= need
                take = jnp.logical_and(hit_j, jnp.logical_not(taken))
                bsp = jnp.where(jnp.broadcast_to(take, (_L,)), bsp2, bsp)
                ncs = jnp.where(take, ncs2, ncs)
                bcnt = jnp.where(take, v_b, bcnt)
                taken = jnp.logical_or(taken, hit_j)
                run = jnp.where(hit_j, run, run + bs)
            return bsp, ncs, bcnt

        pltpu.sync_copy(scores_hbm.at[wid * RPW], keya_v)
        pending = None
        for r in range(RPW):
            if pending is not None:
                pending.wait()
            if r + 1 < RPW:
                pending = pltpu.async_copy(
                    scores_hbm.at[wid * RPW + r + 1],
                    key_bufs[(r + 1) % 2], dma_sem)
            key_v = key_bufs[r % 2]

            def load_key(off, key_v=key_v):
                return lax.bitcast_convert_type(key_v[pl.ds(off, _L)],
                                                jnp.int32)

            hist_clear(_NBINS)

            @plsc.parallel_loop(0, _N // _L, unroll=8)
            def _p1(i):
                iv = load_key(i * _L)
                w = jnp.where(iv < 0, iv ^ _NOSIGN, iv)
                u = w ^ _SIGN
                key_v[pl.ds(i * _L, _L)] = (
                    lax.bitcast_convert_type(u, jnp.float32))
                b1 = lax.shift_right_logical(u, 20)
                plsc.addupdate_scatter(hist_v, [b1], oi)

            b1sp, need1, _ = hist_scan(_NBINS, jnp.int32(_K))

            hist_clear(_NBINS)

            @plsc.parallel_loop(0, _N // _L, unroll=8)
            def _p2(i):
                u = load_key(i * _L)
                m = lax.shift_right_logical(u, 20) == b1sp
                b2 = lax.shift_right_logical(u, 8) & np.int32(0xFFF)
                plsc.addupdate_scatter(hist_v, [b2], oi, mask=m)

            b2sp, need2, _ = hist_scan(_NBINS, need1)

            hist_clear(512)
            p12sp = (b1sp << 12) | b2sp

            @plsc.parallel_loop(0, _N // _L, unroll=8)
            def _p3(i):
                u = load_key(i * _L)
                m = lax.shift_right_logical(u, 8) == p12sp
                b3 = u & np.int32(0xFF)
                plsc.addupdate_scatter(hist_v, [b3], oi, mask=m)
                plsc.addupdate_scatter(hist_v, [b3 + np.int32(256)],
                                       i * _L + lane, mask=m)

            b3sp, need_eq, cnt_eq = hist_scan(256, need2)

            b3s = jnp.max(b3sp)
            ixv = hist_v[pl.ds(256 + ((b3s >> 4) << 4), _L)]
            ohb = lane == (b3sp & np.int32(0xF))
            col1 = jnp.sum(jnp.where(ohb, ixv, zi))

            vk_bits = (p12sp << 8) | b3sp
            wv = vk_bits ^ _SIGN
            fv = jnp.where(wv < 0, wv ^ _NOSIGN, wv)
            vk_f = lax.bitcast_convert_type(fv, jnp.float32)

            need_eq_sp = jnp.broadcast_to(need_eq, (_L,))

            def tw_cond(c):
                return jnp.logical_not(c[2])

            def tw_body(c):
                blk, cntsp, found = c
                base = blk * _BLK
                bcsp = zi
                for j in range(_VPB):
                    u = load_key(base + j * _L)
                    e = u == vk_bits
                    bcsp = bcsp + plsc.all_reduce_population_count(e)
                hit = jnp.any((cntsp + bcsp) >= need_eq_sp)
                blk2 = jnp.where(hit, blk, blk + 1)
                cntsp2 = jnp.where(jnp.broadcast_to(hit, (_L,)),
                                   cntsp, cntsp + bcsp)
                return blk2, cntsp2, hit

            blk_f, cnt_f, _ = lax.while_loop(
                tw_cond, tw_body,
                (jnp.int32(0), zi, cnt_eq == 1))

            base = blk_f * _BLK
            runsp = cnt_f
            big = jnp.full((_L,), np.int32(2**30), jnp.int32)
            irs2 = big
            for j in range(_VPB):
                u = load_key(base + j * _L)
                e = u == vk_bits
                ei = jnp.where(e, oi, zi)
                pr = plsc.cumsum(ei)
                tgt = jnp.logical_and(e, (runsp + pr) == need_eq_sp)
                cnd = jnp.where(tgt, base + j * _L + lane, big)
                irs2 = jnp.minimum(irs2, cnd)
                runsp = runsp + plsc.all_reduce_population_count(e)
            ir_tie = jnp.min(irs2)
            ir_sp = jnp.broadcast_to(
                jnp.where(cnt_eq == 1, col1, ir_tie), (_L,))

            sel = lane == r * 4
            stage_thr = jnp.where(sel, vk_f, stage_thr)
            stage_tie = jnp.where(sel, ir_sp, stage_tie)

        sthr_v[pl.ds(0, _L)] = stage_thr
        stie_v[pl.ds(0, _L)] = stage_tie
        pltpu.sync_copy(sthr_v, thr_hbm.at[wid])
        pltpu.sync_copy(stie_v, tie_hbm.at[wid])

    return sc_select


_sc_select = _build_sc_select()


def _tc_apply_kernel(x_ref, thr_ref, tie_ref, masked_ref, mask_ref):
    x = x_ref[...]
    R, N = x.shape
    vk = thr_ref[:, 0:1]
    tie = tie_ref[:, 0:1]
    col = lax.broadcasted_iota(jnp.int32, (R, N), 1)
    mask = (x > vk) | ((x == vk) & (col <= tie))
    mask_ref[...] = mask
    masked_ref[...] = x * mask.astype(jnp.float32)


@jax.jit
def _run(scores):
    B, N = scores.shape
    thr2, tie2 = _sc_select(scores)
    rpw = B // thr2.shape[0]
    thr = thr2.reshape(B, rpw)
    tie = tie2.reshape(B, rpw)
    R = 64
    masked, mask = pl.pallas_call(
        _tc_apply_kernel,
        grid=(B // R,),
        in_specs=[
            pl.BlockSpec((R, N), lambda b: (b, 0)),
            pl.BlockSpec((R, rpw), lambda b: (b, 0)),
            pl.BlockSpec((R, rpw), lambda b: (b, 0)),
        ],
        out_specs=[
            pl.BlockSpec((R, N), lambda b: (b, 0)),
            pl.BlockSpec((R, N), lambda b: (b, 0)),
        ],
        out_shape=[
            jax.ShapeDtypeStruct((B, N), jnp.float32),
            jax.ShapeDtypeStruct((B, N), jnp.bool_),
        ],
    )(scores, thr, tie)
    return masked, mask


def kernel(scores, k):
    return _run(scores)


# speedup vs baseline: 1.0021x; 1.0021x over previous
import functools

import jax
import jax.numpy as jnp
import numpy as np
from jax import lax
from jax.experimental import pallas as pl
from jax.experimental.pallas import tpu as pltpu
from jax.experimental.pallas import tpu_sc as plsc

_K = 64
_B = 128
_N = 32768
_L = 16
_VPB = 4
_BLK = _L * _VPB
_NBLK = _N // _BLK
_NBINS = 4096
_SIGN = np.int32(-2147483648)
_NOSIGN = np.int32(0x7FFFFFFF)


def _build_sc_select():
    info = plsc.get_sparse_core_info()
    NC, NS = info.num_cores, info.num_subcores
    NW = NC * NS
    RPW = _B // NW

    mesh = plsc.VectorSubcoreMesh(core_axis_name="c", subcore_axis_name="s")

    @functools.partial(
        pl.kernel,
        out_type=[
            jax.ShapeDtypeStruct((NW, _L), jnp.float32),
            jax.ShapeDtypeStruct((NW, _L), jnp.int32),
        ],
        mesh=mesh,
        compiler_params=pltpu.CompilerParams(needs_layout_passes=False),
        scratch_types=[
            pltpu.VMEM((_N,), jnp.float32),
            pltpu.VMEM((_N,), jnp.float32),
            pltpu.VMEM((_NBINS,), jnp.int32),
            pltpu.VMEM((_L,), jnp.float32),
            pltpu.VMEM((_L,), jnp.int32),
            pltpu.SemaphoreType.DMA,
        ],
    )
    def sc_select(scores_hbm, thr_hbm, tie_hbm, keya_v, keyb_v, hist_v,
                  sthr_v, stie_v, dma_sem):
        wid = lax.axis_index("s") * NC + lax.axis_index("c")

        zi = jnp.zeros((_L,), jnp.int32)
        oi = jnp.ones((_L,), jnp.int32)
        lane = lax.broadcasted_iota(jnp.int32, (_L,), 0)
        stage_thr = jnp.zeros((_L,), jnp.float32)
        stage_tie = zi

        key_bufs = [keya_v, keyb_v]

        def hist_clear(nwords):
            @plsc.parallel_loop(0, nwords // _L, unroll=4)
            def _clr(i):
                hist_v[pl.ds(i * _L, _L)] = zi

        def hist_scan(nbins, need):
            def cond(c):
                return jnp.logical_not(c[2])

            def body(c):
                blk4, cum, found = c
                base = blk4 * 4 * _L
                s = hist_v[pl.ds(base, _L)]
                for j in range(1, 4):
                    s = s + hist_v[pl.ds(base + j * _L, _L)]
                bs = jnp.sum(s)
                hit = cum + bs >= need
                blk42 = jnp.where(hit, blk4, blk4 - 1)
                cum2 = jnp.where(hit, cum, cum + bs)
                return blk42, cum2, hit

            init = (jnp.int32(nbins // (4 * _L) - 1), jnp.int32(0),
                    jnp.bool_(False))
            blk4, cum, _ = lax.while_loop(cond, body, init)

            base = blk4 * 4 * _L
            run = cum
            taken = jnp.bool_(False)
            bsp = zi
            ncs = jnp.int32(0)
            bcnt = jnp.int32(0)
            for j in range(3, -1, -1):
                v = hist_v[pl.ds(base + j * _L, _L)]
                bs = jnp.sum(v)
                fw = plsc.cumsum(v)
                cnd = (run + bs - fw + v) >= need
                pc = plsc.all_reduce_population_count(cnd)
                bsp2 = base + j * _L + pc - 1
                oh = lane == (pc - 1)
                fw_b = jnp.sum(jnp.where(oh, fw, zi))
                v_b = jnp.sum(jnp.where(oh, v, zi))
                ncs2 = need - (run + bs - fw_b)
                hit_j = run + bs >= need
                take = jnp.logical_and(hit_j, jnp.logical_not(taken))
                bsp = jnp.where(jnp.broadcast_to(take, (_L,)), bsp2, bsp)
                ncs = jnp.where(take, ncs2, ncs)
                bcnt = jnp.where(take, v_b, bcnt)
                taken = jnp.logical_or(taken, hit_j)
                run = jnp.where(hit_j, run, run + bs)
            return bsp, ncs, bcnt

        pltpu.sync_copy(scores_hbm.at[wid * RPW], keya_v)
        pending = None
        for r in range(RPW):
            if pending is not None:
                pending.wait()
            if r + 1 < RPW:
                pending = pltpu.async_copy(
                    scores_hbm.at[wid * RPW + r + 1],
                    key_bufs[(r + 1) % 2], dma_sem)
            key_v = key_bufs[r % 2]

            def load_key(off, key_v=key_v):
                return lax.bitcast_convert_type(key_v[pl.ds(off, _L)],
                                                jnp.int32)

            hist_clear(_NBINS)

            @plsc.parallel_loop(0, _N // _L, unroll=8)
            def _p1(i):
                iv = load_key(i * _L)
                w = jnp.where(iv < 0, iv ^ _NOSIGN, iv)
                u = w ^ _SIGN
                key_v[pl.ds(i * _L, _L)] = (
                    lax.bitcast_convert_type(u, jnp.float32))
                b1 = lax.shift_right_logical(u, 20)
                plsc.addupdate_scatter(hist_v, [b1], oi)

            b1sp, need1, _ = hist_scan(_NBINS, jnp.int32(_K))

            hist_clear(_NBINS)

            @plsc.parallel_loop(0, _N // _L, unroll=8)
            def _p2(i):
                u = load_key(i * _L)
                m = lax.shift_right_logical(u, 20) == b1sp
                b2 = lax.shift_right_logical(u, 8) & np.int32(0xFFF)
                plsc.addupdate_scatter(hist_v, [b2], oi, mask=m)

            b2sp, need2, _ = hist_scan(_NBINS, need1)

            hist_clear(512)
            p12sp = (b1sp << 12) | b2sp

            @plsc.parallel_loop(0, _N // _L, unroll=8)
            def _p3(i):
                u = load_key(i * _L)
                m = lax.shift_right_logical(u, 8) == p12sp
                b3 = u & np.int32(0xFF)
                plsc.addupdate_scatter(hist_v, [b3], oi, mask=m)
                plsc.addupdate_scatter(hist_v, [b3 + np.int32(256)],
                                       i * _L + lane, mask=m)

            b3sp, need_eq, cnt_eq = hist_scan(256, need2)

            b3s = jnp.max(b3sp)
            ixv = hist_v[pl.ds(256 + ((b3s >> 4) << 4), _L)]
            ohb = lane == (b3sp & np.int32(0xF))
            col1 = jnp.sum(jnp.where(ohb, ixv, zi))

            vk_bits = (p12sp << 8) | b3sp
            wv = vk_bits ^ _SIGN
            fv = jnp.where(wv < 0, wv ^ _NOSIGN, wv)
            vk_f = lax.bitcast_convert_type(fv, jnp.float32)

            need_eq_sp = jnp.broadcast_to(need_eq, (_L,))

            def tw_cond(c):
                return jnp.logical_not(c[2])

            def tw_body(c):
                blk, cntsp, found = c
                base = blk * _BLK
                bcsp = zi
                for j in range(_VPB):
                    u = load_key(base + j * _L)
                    e = u == vk_bits
                    bcsp = bcsp + plsc.all_reduce_population_count(e)
                hit = jnp.any((cntsp + bcsp) >= need_eq_sp)
                blk2 = jnp.where(hit, blk, blk + 1)
                cntsp2 = jnp.where(jnp.broadcast_to(hit, (_L,)),
                                   cntsp, cntsp + bcsp)
                return blk2, cntsp2, hit

            blk_f, cnt_f, _ = lax.while_loop(
                tw_cond, tw_body,
                (jnp.int32(0), zi, cnt_eq == 1))

            base = blk_f * _BLK
            runsp = cnt_f
            big = jnp.full((_L,), np.int32(2**30), jnp.int32)
            irs2 = big
            for j in range(_VPB):
                u = load_key(base + j * _L)
                e = u == vk_bits
                ei = jnp.where(e, oi, zi)
                pr = plsc.cumsum(ei)
                tgt = jnp.logical_and(e, (runsp + pr) == need_eq_sp)
                cnd = jnp.where(tgt, base + j * _L + lane, big)
                irs2 = jnp.minimum(irs2, cnd)
                runsp = runsp + plsc.all_reduce_population_count(e)
            ir_tie = jnp.min(irs2)
            ir_sp = jnp.broadcast_to(
                jnp.where(cnt_eq == 1, col1, ir_tie), (_L,))

            sel = lane == r * 4
            stage_thr = jnp.where(sel, vk_f, stage_thr)
            stage_tie = jnp.where(sel, ir_sp, stage_tie)

        sthr_v[pl.ds(0, _L)] = stage_thr
        stie_v[pl.ds(0, _L)] = stage_tie
        pltpu.sync_copy(sthr_v, thr_hbm.at[wid])
        pltpu.sync_copy(stie_v, tie_hbm.at[wid])

    return sc_select


_sc_select = _build_sc_select()


def _tc_apply_kernel(x_ref, thr_ref, tie_ref, masked_ref, mask_ref):
    x = x_ref[...]
    R, N = x.shape
    vk = thr_ref[:, 0:1]
    tie = tie_ref[:, 0:1]
    col = lax.broadcasted_iota(jnp.int32, (R, N), 1)
    mask = (x > vk) | ((x == vk) & (col <= tie))
    mask_ref[...] = mask
    masked_ref[...] = x * mask.astype(jnp.float32)


@jax.jit
def _run(scores):
    B, N = scores.shape
    thr2, tie2 = _sc_select(scores)
    rpw = B // thr2.shape[0]
    thr = thr2.reshape(B, rpw)
    tie = tie2.reshape(B, rpw)
    R = 32
    masked, mask = pl.pallas_call(
        _tc_apply_kernel,
        grid=(B // R,),
        in_specs=[
            pl.BlockSpec((R, N), lambda b: (b, 0)),
            pl.BlockSpec((R, rpw), lambda b: (b, 0)),
            pl.BlockSpec((R, rpw), lambda b: (b, 0)),
        ],
        out_specs=[
            pl.BlockSpec((R, N), lambda b: (b, 0)),
            pl.BlockSpec((R, N), lambda b: (b, 0)),
        ],
        out_shape=[
            jax.ShapeDtypeStruct((B, N), jnp.float32),
            jax.ShapeDtypeStruct((B, N), jnp.bool_),
        ],
    )(scores, thr, tie)
    return masked, mask


def kernel(scores, k):
    return _run(scores)
